# Initial kernel scaffold; baseline (speedup 1.0000x reference)
import jax
import jax.numpy as jnp
from jax.experimental import pallas as pl


def kernel(x, edge_index, W1, b1, g1, be1, W2, b2, g2, be2, Wm, bm, gm, bem, Wo, bo):
    raise NotImplementedError("write your pallas kernel here")


# speedup vs baseline: 5.2436x; 5.2436x over previous
import functools

import jax
import jax.numpy as jnp
from jax import lax
from jax.experimental import pallas as pl
from jax.experimental.pallas import tpu as pltpu
from jax.experimental.pallas import tpu_sc as plsc

N = 10000
E = 160000
D_IN = 256
H = 512
HM = 256

NC = 2
NS = 16
L = 16

FC = 128
NCH = H // FC
CPC = NCH // NC

B = 128
EP = 163840
NACC = 10240
PAD_DST = 10016

R = 1000
GRID = N // R


def _deg_call(dst32, ones_hbm, zeros_hbm):
  mesh = plsc.VectorSubcoreMesh(core_axis_name="c", subcore_axis_name="s",
                                num_cores=NC, num_subcores=NS)
  nb = EP // 32 // B

  @functools.partial(
      pl.kernel,
      out_type=jax.ShapeDtypeStruct((NC * NACC, FC), jnp.float32),
      mesh=mesh,
      scratch_types=[
          pltpu.VMEM((EP // 32 // B, B), jnp.int32),
          pltpu.VMEM((B, FC), jnp.float32),
          pltpu.VMEM((B, FC), jnp.float32),
          pltpu.VMEM_SHARED((NACC, FC), jnp.float32),
      ],
  )
  def k(dst_hbm, ones_in, zeros_in, out_hbm, idx_v, ones_v, buf_v, acc_sh):
    c = lax.axis_index("c")
    s = lax.axis_index("s")
    tid = c * NS + s
    zr = NACC // NS
    np_ = zr // B
    pltpu.sync_copy(dst_hbm.at[tid], idx_v)
    pltpu.sync_copy(ones_in, ones_v)
    pltpu.sync_copy(zeros_in, buf_v)
    for p in range(np_):
      pltpu.sync_copy(buf_v, acc_sh.at[pl.ds(s * zr + p * B, B)])
    plsc.subcore_barrier()

    def body(j, _):
      pltpu.sync_copy(ones_v, acc_sh.at[idx_v.at[j]], add=True)
      return 0

    lax.fori_loop(0, nb, body, 0)
    plsc.subcore_barrier()
    for p in range(np_):
      pltpu.sync_copy(acc_sh.at[pl.ds(s * zr + p * B, B)], buf_v)
      pltpu.sync_copy(buf_v, out_hbm.at[pl.ds(c * NACC + s * zr + p * B, B)])

  return k(dst32, ones_hbm, zeros_hbm)


def _agg_call(src16, dst16, hp_flat, zeros_hbm):
  mesh = plsc.VectorSubcoreMesh(core_axis_name="c", subcore_axis_name="s",
                                num_cores=NC, num_subcores=NS)
  nb = EP // NS // B

  @functools.partial(
      pl.kernel,
      out_type=jax.ShapeDtypeStruct((NCH * NACC, FC), jnp.float32),
      mesh=mesh,
      scratch_types=[
          pltpu.VMEM((nb, B), jnp.int32),
          pltpu.VMEM((nb, B), jnp.int32),
          pltpu.VMEM((nb, B), jnp.int32),
          pltpu.VMEM((B, FC), jnp.float32),
          pltpu.VMEM_SHARED((NACC, FC), jnp.float32),
          pltpu.SemaphoreType.DMA,
      ],
  )
  def k(src_hbm, dst_hbm, hp_hbm, z_hbm, out_hbm,
        src_v, dst_v, idx_v, rows_v, acc_sh, sem):
    c = lax.axis_index("c")
    s = lax.axis_index("s")
    zr = NACC // NS
    np_ = zr // B
    pltpu.sync_copy(src_hbm.at[s], src_v)
    pltpu.sync_copy(dst_hbm.at[s], dst_v)

    for jc in range(CPC):
      ch = c * CPC + jc
      off = ch * N
      pltpu.sync_copy(z_hbm, rows_v)
      for p in range(np_):
        pltpu.sync_copy(rows_v, acc_sh.at[pl.ds(s * zr + p * B, B)])
      def add_off(r, _):
        for kk in range(B // L):
          idx_v[r, pl.ds(kk * L, L)] = src_v[r, pl.ds(kk * L, L)] + off
        return 0
      lax.fori_loop(0, nb, add_off, 0)
      plsc.subcore_barrier()

      def body(j, _):
        pltpu.async_copy(hp_hbm.at[idx_v.at[j]], rows_v, sem).wait()
        pltpu.sync_copy(rows_v, acc_sh.at[dst_v.at[j]], add=True)
        return 0

      lax.fori_loop(0, nb, body, 0)
      plsc.subcore_barrier()
      for p in range(np_):
        pltpu.sync_copy(acc_sh.at[pl.ds(s * zr + p * B, B)], rows_v)
        pltpu.sync_copy(rows_v,
                        out_hbm.at[pl.ds(ch * NACC + s * zr + p * B, B)])
      plsc.subcore_barrier()

  return k(src16, dst16, hp_flat, zeros_hbm)


def _dis(degp_ref):
  deg = degp_ref[0, :, 0:1] + degp_ref[1, :, 0:1] + 1.0
  return lax.rsqrt(deg)


def _mm_scale_body(x_ref, w_ref, degp_ref, out_ref):
  dis = _dis(degp_ref)
  h = jnp.dot(x_ref[...], w_ref[...], preferred_element_type=jnp.float32)
  hp = h * dis
  for cth in range(NCH):
    out_ref[cth] = hp[:, cth * FC:(cth + 1) * FC]


def _mm_scale(x, W, degp):
  return pl.pallas_call(
      _mm_scale_body,
      grid=(GRID,),
      in_specs=[
          pl.BlockSpec((R, x.shape[1]), lambda i: (i, 0)),
          pl.BlockSpec(W.shape, lambda i: (0, 0)),
          pl.BlockSpec((NC, R, FC), lambda i: (0, i, 0)),
      ],
      out_specs=pl.BlockSpec((NCH, R, FC), lambda i: (0, i, 0)),
      out_shape=jax.ShapeDtypeStruct((NCH, N, FC), jnp.float32),
  )(x, W, degp)


def _post_body(acc_ref, hp_ref, degp_ref, b_ref, pre_ref, st_ref):
  dis = _dis(degp_ref)
  parts = [acc_ref[cth] + hp_ref[cth] for cth in range(NCH)]
  ssum = jnp.concatenate(parts, axis=1)
  pre = ssum * dis + b_ref[...]
  pre_ref[...] = pre

  @pl.when(pl.program_id(0) == 0)
  def _():
    st_ref[...] = jnp.zeros_like(st_ref)

  st_ref[0:1, :] += jnp.sum(pre, axis=0, keepdims=True)
  st_ref[1:2, :] += jnp.sum(pre * pre, axis=0, keepdims=True)


def _post(acc, hp, degp, b):
  return pl.pallas_call(
      _post_body,
      grid=(GRID,),
      in_specs=[
          pl.BlockSpec((NCH, R, FC), lambda i: (0, i, 0)),
          pl.BlockSpec((NCH, R, FC), lambda i: (0, i, 0)),
          pl.BlockSpec((NC, R, FC), lambda i: (0, i, 0)),
          pl.BlockSpec((1, H), lambda i: (0, 0)),
      ],
      out_specs=[
          pl.BlockSpec((R, H), lambda i: (i, 0)),
          pl.BlockSpec((2, H), lambda i: (0, 0)),
      ],
      out_shape=[
          jax.ShapeDtypeStruct((N, H), jnp.float32),
          jax.ShapeDtypeStruct((2, H), jnp.float32),
      ],
  )(acc, hp, degp, b)


def _bn(pre, st_ref, g_ref, be_ref):
  mean = st_ref[0:1, :] * (1.0 / N)
  ex2 = st_ref[1:2, :] * (1.0 / N)
  var = ex2 - mean * mean
  inv = lax.rsqrt(var + 1e-5)
  return jnp.maximum((pre - mean) * inv * g_ref[...] + be_ref[...], 0.0)


def _bn_mm_scale_body(pre_ref, st_ref, g_ref, be_ref, w_ref, degp_ref, out_ref):
  dis = _dis(degp_ref)
  h = _bn(pre_ref[...], st_ref, g_ref, be_ref)
  hh = jnp.dot(h, w_ref[...], preferred_element_type=jnp.float32)
  hp = hh * dis
  for cth in range(NCH):
    out_ref[cth] = hp[:, cth * FC:(cth + 1) * FC]


def _bn_mm_scale(pre, st, g, be, W, degp):
  return pl.pallas_call(
      _bn_mm_scale_body,
      grid=(GRID,),
      in_specs=[
          pl.BlockSpec((R, H), lambda i: (i, 0)),
          pl.BlockSpec((2, H), lambda i: (0, 0)),
          pl.BlockSpec((1, H), lambda i: (0, 0)),
          pl.BlockSpec((1, H), lambda i: (0, 0)),
          pl.BlockSpec(W.shape, lambda i: (0, 0)),
          pl.BlockSpec((NC, R, FC), lambda i: (0, i, 0)),
      ],
      out_specs=pl.BlockSpec((NCH, R, FC), lambda i: (0, i, 0)),
      out_shape=jax.ShapeDtypeStruct((NCH, N, FC), jnp.float32),
  )(pre, st, g, be, W, degp)


def _bn_mm_bias_body(pre_ref, st_ref, g_ref, be_ref, w_ref, b_ref,
                     out_ref, st3_ref):
  h = _bn(pre_ref[...], st_ref, g_ref, be_ref)
  o = jnp.dot(h, w_ref[...], preferred_element_type=jnp.float32) + b_ref[...]
  out_ref[...] = o

  @pl.when(pl.program_id(0) == 0)
  def _():
    st3_ref[...] = jnp.zeros_like(st3_ref)

  st3_ref[0:1, :] += jnp.sum(o, axis=0, keepdims=True)
  st3_ref[1:2, :] += jnp.sum(o * o, axis=0, keepdims=True)


def _bn_mm_bias(pre, st, g, be, W, b):
  return pl.pallas_call(
      _bn_mm_bias_body,
      grid=(GRID,),
      in_specs=[
          pl.BlockSpec((R, H), lambda i: (i, 0)),
          pl.BlockSpec((2, H), lambda i: (0, 0)),
          pl.BlockSpec((1, H), lambda i: (0, 0)),
          pl.BlockSpec((1, H), lambda i: (0, 0)),
          pl.BlockSpec(W.shape, lambda i: (0, 0)),
          pl.BlockSpec((1, HM), lambda i: (0, 0)),
      ],
      out_specs=[
          pl.BlockSpec((R, HM), lambda i: (i, 0)),
          pl.BlockSpec((2, HM), lambda i: (0, 0)),
      ],
      out_shape=[
          jax.ShapeDtypeStruct((N, HM), jnp.float32),
          jax.ShapeDtypeStruct((2, HM), jnp.float32),
      ],
  )(pre, st, g, be, W, b)


def _final_body(pre_ref, st_ref, g_ref, be_ref, w_ref, b_ref, out_ref):
  h = _bn(pre_ref[...], st_ref, g_ref, be_ref)
  o = jnp.dot(h, w_ref[...], preferred_element_type=jnp.float32) + b_ref[...]
  out_ref[...] = jax.nn.sigmoid(o)


def _final(pre, st, g, be, W, b):
  return pl.pallas_call(
      _final_body,
      grid=(GRID,),
      in_specs=[
          pl.BlockSpec((R, HM), lambda i: (i, 0)),
          pl.BlockSpec((2, HM), lambda i: (0, 0)),
          pl.BlockSpec((1, HM), lambda i: (0, 0)),
          pl.BlockSpec((1, HM), lambda i: (0, 0)),
          pl.BlockSpec(W.shape, lambda i: (0, 0)),
          pl.BlockSpec((1, 1), lambda i: (0, 0)),
      ],
      out_specs=pl.BlockSpec((R, 1), lambda i: (i, 0)),
      out_shape=jax.ShapeDtypeStruct((N, 1), jnp.float32),
  )(pre, st, g, be, W, b)


def kernel(x, edge_index, W1, b1, g1, be1, W2, b2, g2, be2, Wm, bm, gm, bem,
           Wo, bo):
  src = edge_index[0]
  dst = edge_index[1]
  pad = EP - E
  src_p = jnp.concatenate([src, jnp.zeros((pad,), jnp.int32)])
  dst_p = jnp.concatenate([dst, jnp.full((pad,), PAD_DST, jnp.int32)])
  src16 = src_p.reshape(NS, EP // NS // B, B)
  dst16 = dst_p.reshape(NS, EP // NS // B, B)
  dst32 = dst_p.reshape(32, EP // 32 // B, B)

  ones_hbm = jnp.ones((B, FC), jnp.float32)
  zeros_hbm = jnp.zeros((B, FC), jnp.float32)

  degp = _deg_call(dst32, ones_hbm, zeros_hbm).reshape(NC, NACC, FC)

  hp1 = _mm_scale(x, W1, degp)
  acc1 = _agg_call(src16, dst16, hp1.reshape(NCH * N, FC), zeros_hbm)
  pre1, st1 = _post(acc1.reshape(NCH, NACC, FC), hp1, degp, b1.reshape(1, H))

  hp2 = _bn_mm_scale(pre1, st1, g1.reshape(1, H), be1.reshape(1, H), W2, degp)
  acc2 = _agg_call(src16, dst16, hp2.reshape(NCH * N, FC), zeros_hbm)
  pre2, st2 = _post(acc2.reshape(NCH, NACC, FC), hp2, degp, b2.reshape(1, H))

  pre3, st3 = _bn_mm_bias(pre2, st2, g2.reshape(1, H), be2.reshape(1, H),
                          Wm, bm.reshape(1, HM))
  return _final(pre3, st3, gm.reshape(1, HM), bem.reshape(1, HM),
                Wo, bo.reshape(1, 1))
